# baseline (speedup 1.0000x reference)
import functools

import jax
import jax.numpy as jnp
import numpy as np
from jax import lax
from jax.experimental import pallas as pl
from jax.experimental.pallas import tpu as pltpu
from jax.experimental.pallas import tpu_sc as plsc

N = 10000
E = 320000
D = 128
L = 2
S = 4
FF = 512
H = 8
DK = 16
RB = 1000
GRID = N // RB
F32 = jnp.float32


def _ln(y, g, b):
    m = jnp.mean(y, -1, keepdims=True)
    v = jnp.mean((y - m) ** 2, -1, keepdims=True)
    return (y - m) * lax.rsqrt(v + 1e-5) * g + b


def _pre_body(x_ref, w_ref, bse_ref, o_ref):
    o_ref[...] = (jnp.dot(x_ref[...], w_ref[...], preferred_element_type=F32)
                  + bse_ref[...])


def _pre(x, wt, bse):
    return pl.pallas_call(
        _pre_body,
        grid=(GRID,),
        in_specs=[
            pl.BlockSpec((RB, D), lambda i: (i, 0)),
            pl.BlockSpec((D, D), lambda i: (0, 0)),
            pl.BlockSpec((1, D), lambda i: (0, 0)),
        ],
        out_specs=pl.BlockSpec((RB, D), lambda i: (i, 0)),
        out_shape=jax.ShapeDtypeStruct((N, D), F32),
    )(x, wt, bse)


QW = 136
KVW = 264


def _qkv_body(h_ref, wq_ref, wkv_ref, q_ref, kv_ref):
    h = h_ref[...]
    q_ref[:, :D] = jnp.dot(h, wq_ref[...], preferred_element_type=F32)
    kv_ref[:, :2 * D] = jnp.dot(h, wkv_ref[...], preferred_element_type=F32)


def _qkv(h, wqt, wkvt):
    return pl.pallas_call(
        _qkv_body,
        grid=(GRID,),
        in_specs=[
            pl.BlockSpec((RB, D), lambda i: (i, 0)),
            pl.BlockSpec((D, D), lambda i: (0, 0)),
            pl.BlockSpec((D, 2 * D), lambda i: (0, 0)),
        ],
        out_specs=[
            pl.BlockSpec((RB, QW), lambda i: (i, 0)),
            pl.BlockSpec((RB, KVW), lambda i: (i, 0)),
        ],
        out_shape=[
            jax.ShapeDtypeStruct((N, QW), F32),
            jax.ShapeDtypeStruct((N, KVW), F32),
        ],
    )(h, wqt, wkvt)


_SQ2I = np.float32(1.0 / np.sqrt(2.0))


def _post_body(p_ref, h_ref, expand_ref, wo_ref, bo_ref, w1_ref, b1_ref,
               w2_ref, b2_ref, l1g_ref, l1b_ref, l2g_ref, l2b_ref, o_ref):
    num = p_ref[0, :, :D] + p_ref[1, :, :D]
    den = p_ref[0, :, D:D + H] + p_ref[1, :, D:D + H]
    rec = 1.0 / (den + 1e-16)
    attn = num * jnp.dot(rec, expand_ref[...], preferred_element_type=F32)
    y = jnp.dot(attn, wo_ref[...], preferred_element_type=F32) + bo_ref[...] + h_ref[...]
    y = _ln(y, l1g_ref[...], l1b_ref[...])
    t = jnp.dot(y, w1_ref[...], preferred_element_type=F32) + b1_ref[...]
    t = 0.5 * t * (1.0 + lax.erf(t * _SQ2I))
    f = jnp.dot(t, w2_ref[...], preferred_element_type=F32) + b2_ref[...]
    o_ref[...] = _ln(y + f, l2g_ref[...], l2b_ref[...])


def _post(partials, h, expand, wot, bo, w1t, b1, w2t, b2, l1g, l1b, l2g, l2b):
    full = lambda a, b_: pl.BlockSpec((a, b_), lambda i: (0, 0))
    return pl.pallas_call(
        _post_body,
        grid=(GRID,),
        in_specs=[
            pl.BlockSpec((2, RB, D + 2 * H), lambda i: (0, i, 0)),
            pl.BlockSpec((RB, D), lambda i: (i, 0)),
            full(H, D), full(D, D), full(1, D),
            full(D, FF), full(1, FF), full(FF, D), full(1, D),
            full(1, D), full(1, D), full(1, D), full(1, D),
        ],
        out_specs=pl.BlockSpec((RB, D), lambda i: (i, 0)),
        out_shape=jax.ShapeDtypeStruct((N, D), F32),
    )(partials, h, expand, wot, bo, w1t, b1, w2t, b2, l1g, l1b, l2g, l2b)


def _fusion_body(h0_ref, h1_ref, h2_ref, h3_ref, fw_ref, fb_ref, g_ref, b_ref,
                 o_ref):
    s = jnp.dot(h0_ref[...], fw_ref[0:D, :], preferred_element_type=F32)
    s += jnp.dot(h1_ref[...], fw_ref[D:2 * D, :], preferred_element_type=F32)
    s += jnp.dot(h2_ref[...], fw_ref[2 * D:3 * D, :], preferred_element_type=F32)
    s += jnp.dot(h3_ref[...], fw_ref[3 * D:4 * D, :], preferred_element_type=F32)
    s += fb_ref[...]
    o_ref[...] = _ln(s, g_ref[...], b_ref[...])


def _fusion(hs, fwt, fb, g, b):
    full = lambda a, b_: pl.BlockSpec((a, b_), lambda i: (0, 0))
    return pl.pallas_call(
        _fusion_body,
        grid=(GRID,),
        in_specs=[pl.BlockSpec((RB, D), lambda i: (i, 0))] * 4 + [
            full(S * D, D), full(1, D), full(1, D), full(1, D)],
        out_specs=pl.BlockSpec((RB, D), lambda i: (i, 0)),
        out_shape=jax.ShapeDtypeStruct((N, D), F32),
    )(*hs, fwt, fb, g, b)


NC = 2
NS = 16
NW = NC * NS
C = 64
PW = D + 2 * H
SP = 512
CPS = SP // C
NSP = E // SP
SP_BASE = NSP // NW
SP_REM = NSP % NW
NP_ = 10240
RPT = NP_ // NS
_SCALE_ATTN = np.float32(1.0 / np.sqrt(DK))


def _edge_partials(q, kv, src, dst, scale, fcl, zblk):
    mesh = plsc.VectorSubcoreMesh(core_axis_name="c", subcore_axis_name="s")

    @functools.partial(
        pl.kernel,
        out_type=jax.ShapeDtypeStruct((NC, NP_, PW), F32),
        mesh=mesh,
        compiler_params=pltpu.CompilerParams(needs_layout_passes=False,
                                             use_tc_tiling_on_sc=False),
        scratch_types=[
            pltpu.VMEM((SP,), jnp.int32),
            pltpu.VMEM((CPS, C), jnp.int32),
            pltpu.VMEM((SP,), F32),
            pltpu.VMEM((SP,), F32),
            pltpu.VMEM((C, QW), F32),
            pltpu.VMEM((C, KVW), F32),
            pltpu.VMEM((C, PW), F32),
            pltpu.VMEM_SHARED((NP_, PW), F32),
            pltpu.SemaphoreType.DMA,
            pltpu.SemaphoreType.DMA,
            pltpu.SemaphoreType.DMA,
            pltpu.SemaphoreType.DMA,
        ],
    )
    def edge_kernel(q_hbm, kv_hbm, src_hbm, dst2_hbm, scale_hbm, fcl_hbm,
                    z_hbm, out_hbm, src_v, dst3, scale_v, fcl_v, qrows,
                    kvrows, wvex, acc, sem1, sem2, sem3, sem4):
        cid = lax.axis_index("c")
        sid = lax.axis_index("s")
        wid = sid * NC + cid
        pltpu.sync_copy(z_hbm, wvex)
        row0 = sid * RPT
        for jz in range(RPT // C):
            pltpu.sync_copy(z_hbm, acc.at[pl.ds(row0 + jz * C, C)])
        plsc.subcore_barrier()

        nsp = jnp.where(wid < SP_REM, SP_BASE + 1, SP_BASE)

        def sp_body(i, carry):
            t = wid + i * NW
            base = t * SP
            pltpu.sync_copy(src_hbm.at[pl.ds(base, SP)], src_v)
            pltpu.sync_copy(dst2_hbm.at[pl.ds(t * CPS, CPS)], dst3)
            pltpu.sync_copy(scale_hbm.at[pl.ds(base, SP)], scale_v)
            pltpu.sync_copy(fcl_hbm.at[pl.ds(base, SP)], fcl_v)

            def chunk_body(c, ccarry):
                cb = c * C
                cp1 = pltpu.async_copy(q_hbm.at[dst3.at[c]], qrows, sem1)
                cp2 = pltpu.async_copy(
                    kv_hbm.at[src_v.at[pl.ds(cb, C // 2)]],
                    kvrows.at[pl.ds(0, C // 2)], sem2)
                cp3 = pltpu.async_copy(
                    kv_hbm.at[src_v.at[pl.ds(cb + C // 2, C // 2)]],
                    kvrows.at[pl.ds(C // 2, C // 2)], sem3)
                cp1.wait()
                cp2.wait()
                cp3.wait()

                def group(g, gcarry):
                    eidx = g * 16 + lax.iota(jnp.int32, 16)
                    scl = scale_v[pl.ds(cb + g * 16, 16)]
                    fcv = fcl_v[pl.ds(cb + g * 16, 16)]
                    for h in range(H):
                        cbase = h * DK
                        logit = jnp.zeros((16,), F32)
                        for j in range(DK):
                            colv = jnp.full((16,), cbase + j, jnp.int32)
                            qv = plsc.load_gather(qrows, [eidx, colv])
                            kj = plsc.load_gather(kvrows, [eidx, colv])
                            logit = logit + qv * kj
                        ex = jnp.exp(logit * scl + fcv)
                        plsc.store_scatter(
                            wvex, [eidx, jnp.full((16,), D + h, jnp.int32)], ex)
                        for j in range(DK):
                            vv = plsc.load_gather(
                                kvrows,
                                [eidx, jnp.full((16,), D + cbase + j, jnp.int32)])
                            plsc.store_scatter(
                                wvex,
                                [eidx, jnp.full((16,), cbase + j, jnp.int32)],
                                ex * vv)
                    return gcarry

                @plsc.parallel_loop(0, C // 16, 1, unroll=4)
                def group(g):
                    eidx = g * 16 + lax.iota(jnp.int32, 16)
                    scl = scale_v[pl.ds(cb + g * 16, 16)]
                    fcv = fcl_v[pl.ds(cb + g * 16, 16)]
                    for h in range(H):
                        cbase = h * DK
                        part = [jnp.zeros((16,), F32) for _ in range(4)]
                        for j in range(DK):
                            colv = jnp.full((16,), cbase + j, jnp.int32)
                            qv = plsc.load_gather(qrows, [eidx, colv])
                            kj = plsc.load_gather(kvrows, [eidx, colv])
                            part[j % 4] = part[j % 4] + qv * kj
                        logit = (part[0] + part[1]) + (part[2] + part[3])
                        ex = jnp.exp(logit * scl + fcv)
                        plsc.store_scatter(
                            wvex, [eidx, jnp.full((16,), D + h, jnp.int32)], ex)
                        for j in range(DK):
                            vv = plsc.load_gather(
                                kvrows,
                                [eidx, jnp.full((16,), D + cbase + j, jnp.int32)])
                            plsc.store_scatter(
                                wvex,
                                [eidx, jnp.full((16,), cbase + j, jnp.int32)],
                                ex * vv)

                pltpu.sync_copy(wvex, acc.at[dst3.at[c]], add=True)
                return ccarry

            lax.fori_loop(0, CPS, chunk_body, 0)
            return carry

        lax.fori_loop(0, nsp, sp_body, 0)
        plsc.subcore_barrier()
        for jz in range(RPT // C):
            pltpu.sync_copy(acc.at[pl.ds(row0 + jz * C, C)],
                            out_hbm.at[cid, pl.ds(row0 + jz * C, C)])

    return edge_kernel(q, kv, src, dst.reshape(E // C, C), scale, fcl, zblk)


def kernel(x, edge_index_list, sc_mask_list, fc_weights_list, input_proj_W,
           input_proj_b, stage_embed, WQ, WK, WV, WO, bO, ln1_g, ln1_b, ln2_g,
           ln2_b, W1, b1, W2, b2, fc_lambda, fusion_W, fusion_b, norm_g,
           norm_b):
    expand = jnp.kron(jnp.eye(H, dtype=F32), jnp.ones((1, DK), F32))
    ipwt = input_proj_W.T
    zblk = jnp.zeros((C, PW), F32)
    outs = []
    for k in range(S):
        bse = (input_proj_b + stage_embed[k]).reshape(1, D)
        h = _pre(x, ipwt, bse)
        src = edge_index_list[k, 0]
        dst = edge_index_list[k, 1]
        scale = sc_mask_list[k].astype(F32) * _SCALE_ATTN
        for l in range(L):
            q, kv = _qkv(h, WQ[l].T, jnp.concatenate([WK[l].T, WV[l].T], axis=1))
            partials = _edge_partials(q, kv, src, dst, scale,
                                      fc_lambda[l] * fc_weights_list[k], zblk)
            h = _post(partials, h, expand, WO[l].T, bO[l].reshape(1, D),
                      W1[l].T, b1[l].reshape(1, FF), W2[l].T,
                      b2[l].reshape(1, D), ln1_g[l].reshape(1, D),
                      ln1_b[l].reshape(1, D), ln2_g[l].reshape(1, D),
                      ln2_b[l].reshape(1, D))
        outs.append(h)
    return _fusion(outs, fusion_W.T, fusion_b.reshape(1, D),
                   norm_g.reshape(1, D), norm_b.reshape(1, D))


# speedup vs baseline: 1.5083x; 1.5083x over previous
import functools

import jax
import jax.numpy as jnp
import numpy as np
from jax import lax
from jax.experimental import pallas as pl
from jax.experimental.pallas import tpu as pltpu
from jax.experimental.pallas import tpu_sc as plsc

N = 10000
E = 320000
D = 128
L = 2
S = 4
FF = 512
H = 8
DK = 16
RB = 1000
GRID = N // RB
F32 = jnp.float32


def _ln(y, g, b):
    m = jnp.mean(y, -1, keepdims=True)
    v = jnp.mean((y - m) ** 2, -1, keepdims=True)
    return (y - m) * lax.rsqrt(v + 1e-5) * g + b


def _pre_body(x_ref, w_ref, bse_ref, o_ref):
    o_ref[...] = (jnp.dot(x_ref[...], w_ref[...], preferred_element_type=F32)
                  + bse_ref[...])


def _pre(x, wt, bse):
    return pl.pallas_call(
        _pre_body,
        grid=(GRID,),
        in_specs=[
            pl.BlockSpec((RB, D), lambda i: (i, 0)),
            pl.BlockSpec((D, D), lambda i: (0, 0)),
            pl.BlockSpec((1, D), lambda i: (0, 0)),
        ],
        out_specs=pl.BlockSpec((RB, D), lambda i: (i, 0)),
        out_shape=jax.ShapeDtypeStruct((N, D), F32),
    )(x, wt, bse)


QW = 136
KVW = 264


def _qkv_body(h_ref, wq_ref, wkv_ref, q_ref, kv_ref):
    h = h_ref[...]
    q_ref[:, :D] = jnp.dot(h, wq_ref[...], preferred_element_type=F32)
    kv_ref[:, :2 * D] = jnp.dot(h, wkv_ref[...], preferred_element_type=F32)


def _qkv(h, wqt, wkvt):
    return pl.pallas_call(
        _qkv_body,
        grid=(GRID,),
        in_specs=[
            pl.BlockSpec((RB, D), lambda i: (i, 0)),
            pl.BlockSpec((D, D), lambda i: (0, 0)),
            pl.BlockSpec((D, 2 * D), lambda i: (0, 0)),
        ],
        out_specs=[
            pl.BlockSpec((RB, QW), lambda i: (i, 0)),
            pl.BlockSpec((RB, KVW), lambda i: (i, 0)),
        ],
        out_shape=[
            jax.ShapeDtypeStruct((N, QW), F32),
            jax.ShapeDtypeStruct((N, KVW), F32),
        ],
    )(h, wqt, wkvt)


_SQ2I = np.float32(1.0 / np.sqrt(2.0))


def _post_body(p_ref, h_ref, expand_ref, wo_ref, bo_ref, w1_ref, b1_ref,
               w2_ref, b2_ref, l1g_ref, l1b_ref, l2g_ref, l2b_ref, o_ref):
    num = p_ref[0, :, :D] + p_ref[1, :, :D]
    den = p_ref[0, :, D:D + H] + p_ref[1, :, D:D + H]
    rec = 1.0 / (den + 1e-16)
    attn = num * jnp.dot(rec, expand_ref[...], preferred_element_type=F32)
    y = jnp.dot(attn, wo_ref[...], preferred_element_type=F32) + bo_ref[...] + h_ref[...]
    y = _ln(y, l1g_ref[...], l1b_ref[...])
    t = jnp.dot(y, w1_ref[...], preferred_element_type=F32) + b1_ref[...]
    t = 0.5 * t * (1.0 + lax.erf(t * _SQ2I))
    f = jnp.dot(t, w2_ref[...], preferred_element_type=F32) + b2_ref[...]
    o_ref[...] = _ln(y + f, l2g_ref[...], l2b_ref[...])


def _post(partials, h, expand, wot, bo, w1t, b1, w2t, b2, l1g, l1b, l2g, l2b):
    full = lambda a, b_: pl.BlockSpec((a, b_), lambda i: (0, 0))
    return pl.pallas_call(
        _post_body,
        grid=(GRID,),
        in_specs=[
            pl.BlockSpec((2, RB, D + 2 * H), lambda i: (0, i, 0)),
            pl.BlockSpec((RB, D), lambda i: (i, 0)),
            full(H, D), full(D, D), full(1, D),
            full(D, FF), full(1, FF), full(FF, D), full(1, D),
            full(1, D), full(1, D), full(1, D), full(1, D),
        ],
        out_specs=pl.BlockSpec((RB, D), lambda i: (i, 0)),
        out_shape=jax.ShapeDtypeStruct((N, D), F32),
    )(partials, h, expand, wot, bo, w1t, b1, w2t, b2, l1g, l1b, l2g, l2b)


def _fusion_body(h0_ref, h1_ref, h2_ref, h3_ref, fw_ref, fb_ref, g_ref, b_ref,
                 o_ref):
    s = jnp.dot(h0_ref[...], fw_ref[0:D, :], preferred_element_type=F32)
    s += jnp.dot(h1_ref[...], fw_ref[D:2 * D, :], preferred_element_type=F32)
    s += jnp.dot(h2_ref[...], fw_ref[2 * D:3 * D, :], preferred_element_type=F32)
    s += jnp.dot(h3_ref[...], fw_ref[3 * D:4 * D, :], preferred_element_type=F32)
    s += fb_ref[...]
    o_ref[...] = _ln(s, g_ref[...], b_ref[...])


def _fusion(hs, fwt, fb, g, b):
    full = lambda a, b_: pl.BlockSpec((a, b_), lambda i: (0, 0))
    return pl.pallas_call(
        _fusion_body,
        grid=(GRID,),
        in_specs=[pl.BlockSpec((RB, D), lambda i: (i, 0))] * 4 + [
            full(S * D, D), full(1, D), full(1, D), full(1, D)],
        out_specs=pl.BlockSpec((RB, D), lambda i: (i, 0)),
        out_shape=jax.ShapeDtypeStruct((N, D), F32),
    )(*hs, fwt, fb, g, b)


NC = 2
NS = 16
NW = NC * NS
C = 32
PW = D + 2 * H
SP = 512
CPS = SP // C
NSP = E // SP
SP_BASE = NSP // NW
SP_REM = NSP % NW
NP_ = 10240
RPT = NP_ // NS
_SCALE_ATTN = np.float32(1.0 / np.sqrt(DK))


def _edge_partials(q, kv, src, dst, scale, fcl, zblk):
    mesh = plsc.VectorSubcoreMesh(core_axis_name="c", subcore_axis_name="s")

    @functools.partial(
        pl.kernel,
        out_type=jax.ShapeDtypeStruct((NC, NP_, PW), F32),
        mesh=mesh,
        compiler_params=pltpu.CompilerParams(needs_layout_passes=False,
                                             use_tc_tiling_on_sc=False),
        scratch_types=[
            pltpu.VMEM((SP,), jnp.int32),
            pltpu.VMEM((CPS, C), jnp.int32),
            pltpu.VMEM((SP,), F32),
            pltpu.VMEM((SP,), F32),
            pltpu.VMEM((C, QW), F32),
            pltpu.VMEM((C, QW), F32),
            pltpu.VMEM((C, KVW), F32),
            pltpu.VMEM((C, KVW), F32),
            pltpu.VMEM((C, PW), F32),
            pltpu.VMEM_SHARED((NP_, PW), F32),
            pltpu.SemaphoreType.DMA,
            pltpu.SemaphoreType.DMA,
            pltpu.SemaphoreType.DMA,
            pltpu.SemaphoreType.DMA,
        ],
    )
    def edge_kernel(q_hbm, kv_hbm, src_hbm, dst2_hbm, scale_hbm, fcl_hbm,
                    z_hbm, out_hbm, src_v, dst3, scale_v, fcl_v, qrows0,
                    qrows1, kvrows0, kvrows1, wvex, acc, semq0, semq1,
                    semk0, semk1):
        cid = lax.axis_index("c")
        sid = lax.axis_index("s")
        wid = sid * NC + cid
        pltpu.sync_copy(z_hbm, wvex)
        row0 = sid * RPT
        for jz in range(RPT // C):
            pltpu.sync_copy(z_hbm, acc.at[pl.ds(row0 + jz * C, C)])
        plsc.subcore_barrier()

        nsp = jnp.where(wid < SP_REM, SP_BASE + 1, SP_BASE)

        qbufs = (qrows0, qrows1)
        kbufs = (kvrows0, kvrows1)
        qsems = (semq0, semq1)
        ksems = (semk0, semk1)

        def issue(cc, b):
            pltpu.async_copy(q_hbm.at[dst3.at[cc]], qbufs[b], qsems[b])
            pltpu.async_copy(kv_hbm.at[src_v.at[pl.ds(cc * C, C)]],
                             kbufs[b], ksems[b])

        def wait(b):
            pltpu.make_async_copy(q_hbm.at[pl.ds(0, C)], qbufs[b],
                                  qsems[b]).wait()
            pltpu.make_async_copy(kv_hbm.at[pl.ds(0, C)], kbufs[b],
                                  ksems[b]).wait()

        def compute(cc, b):
            qrows = qbufs[b]
            kvrows = kbufs[b]
            cb = cc * C

            @plsc.parallel_loop(0, C // 16, 1, unroll=1)
            def group(g):
                eidx = g * 16 + lax.iota(jnp.int32, 16)
                scl = scale_v[pl.ds(cb + g * 16, 16)]
                fcv = fcl_v[pl.ds(cb + g * 16, 16)]
                for h in range(H):
                    cbase = h * DK
                    part = [jnp.zeros((16,), F32) for _ in range(4)]
                    for j in range(DK):
                        colv = jnp.full((16,), cbase + j, jnp.int32)
                        qv = plsc.load_gather(qrows, [eidx, colv])
                        kj = plsc.load_gather(kvrows, [eidx, colv])
                        part[j % 4] = part[j % 4] + qv * kj
                    logit = (part[0] + part[1]) + (part[2] + part[3])
                    ex = jnp.exp(logit * scl + fcv)
                    plsc.store_scatter(
                        wvex, [eidx, jnp.full((16,), D + h, jnp.int32)], ex)
                    for j in range(DK):
                        vv = plsc.load_gather(
                            kvrows,
                            [eidx, jnp.full((16,), D + cbase + j, jnp.int32)])
                        plsc.store_scatter(
                            wvex,
                            [eidx, jnp.full((16,), cbase + j, jnp.int32)],
                            ex * vv)

            pltpu.sync_copy(wvex, acc.at[dst3.at[cc]], add=True)

        def sp_body(i, carry):
            t = wid + i * NW
            base = t * SP
            pltpu.sync_copy(src_hbm.at[pl.ds(base, SP)], src_v)
            pltpu.sync_copy(dst2_hbm.at[pl.ds(t * CPS, CPS)], dst3)
            pltpu.sync_copy(scale_hbm.at[pl.ds(base, SP)], scale_v)
            pltpu.sync_copy(fcl_hbm.at[pl.ds(base, SP)], fcl_v)
            issue(0, 0)

            def pair_body(c2, pcarry):
                for b in range(2):
                    cc = c2 * 2 + b

                    @pl.when(cc + 1 < CPS)
                    def _():
                        issue(cc + 1, 1 - b)

                    wait(b)
                    compute(cc, b)
                return pcarry

            lax.fori_loop(0, CPS // 2, pair_body, 0)
            return carry

        lax.fori_loop(0, nsp, sp_body, 0)
        plsc.subcore_barrier()
        for jz in range(RPT // C):
            pltpu.sync_copy(acc.at[pl.ds(row0 + jz * C, C)],
                            out_hbm.at[cid, pl.ds(row0 + jz * C, C)])

    return edge_kernel(q, kv, src, dst.reshape(E // C, C), scale, fcl, zblk)


def kernel(x, edge_index_list, sc_mask_list, fc_weights_list, input_proj_W,
           input_proj_b, stage_embed, WQ, WK, WV, WO, bO, ln1_g, ln1_b, ln2_g,
           ln2_b, W1, b1, W2, b2, fc_lambda, fusion_W, fusion_b, norm_g,
           norm_b):
    expand = jnp.kron(jnp.eye(H, dtype=F32), jnp.ones((1, DK), F32))
    ipwt = input_proj_W.T
    zblk = jnp.zeros((C, PW), F32)
    outs = []
    for k in range(S):
        bse = (input_proj_b + stage_embed[k]).reshape(1, D)
        h = _pre(x, ipwt, bse)
        src = edge_index_list[k, 0]
        dst = edge_index_list[k, 1]
        scale = sc_mask_list[k].astype(F32) * _SCALE_ATTN
        for l in range(L):
            q, kv = _qkv(h, WQ[l].T, jnp.concatenate([WK[l].T, WV[l].T], axis=1))
            partials = _edge_partials(q, kv, src, dst, scale,
                                      fc_lambda[l] * fc_weights_list[k], zblk)
            h = _post(partials, h, expand, WO[l].T, bO[l].reshape(1, D),
                      W1[l].T, b1[l].reshape(1, FF), W2[l].T,
                      b2[l].reshape(1, D), ln1_g[l].reshape(1, D),
                      ln1_b[l].reshape(1, D), ln2_g[l].reshape(1, D),
                      ln2_b[l].reshape(1, D))
        outs.append(h)
    return _fusion(outs, fusion_W.T, fusion_b.reshape(1, D),
                   norm_g.reshape(1, D), norm_b.reshape(1, D))


# speedup vs baseline: 3.0559x; 2.0260x over previous
import functools

import jax
import jax.numpy as jnp
import numpy as np
from jax import lax
from jax.experimental import pallas as pl
from jax.experimental.pallas import tpu as pltpu
from jax.experimental.pallas import tpu_sc as plsc

N = 10000
E = 320000
D = 128
L = 2
S = 4
FF = 512
H = 8
DK = 16
RB = 1000
GRID = N // RB
F32 = jnp.float32


def _ln(y, g, b):
    m = jnp.mean(y, -1, keepdims=True)
    v = jnp.mean((y - m) ** 2, -1, keepdims=True)
    return (y - m) * lax.rsqrt(v + 1e-5) * g + b


def _pre_body(x_ref, w_ref, bse_ref, o_ref):
    o_ref[...] = (jnp.dot(x_ref[...], w_ref[...], preferred_element_type=F32)
                  + bse_ref[...])


def _pre(x, wt, bse):
    return pl.pallas_call(
        _pre_body,
        grid=(GRID,),
        in_specs=[
            pl.BlockSpec((RB, D), lambda i: (i, 0)),
            pl.BlockSpec((D, D), lambda i: (0, 0)),
            pl.BlockSpec((1, D), lambda i: (0, 0)),
        ],
        out_specs=pl.BlockSpec((RB, D), lambda i: (i, 0)),
        out_shape=jax.ShapeDtypeStruct((N, D), F32),
    )(x, wt, bse)


QW = 136
KVW = 264


def _qkv_body(h_ref, wq_ref, wkv_ref, q_ref, kv_ref):
    h = h_ref[...]
    q_ref[:, :D] = jnp.dot(h, wq_ref[...], preferred_element_type=F32)
    kv_ref[:, :2 * D] = jnp.dot(h, wkv_ref[...], preferred_element_type=F32)


def _qkv(h, wqt, wkvt):
    return pl.pallas_call(
        _qkv_body,
        grid=(GRID,),
        in_specs=[
            pl.BlockSpec((RB, D), lambda i: (i, 0)),
            pl.BlockSpec((D, D), lambda i: (0, 0)),
            pl.BlockSpec((D, 2 * D), lambda i: (0, 0)),
        ],
        out_specs=[
            pl.BlockSpec((RB, QW), lambda i: (i, 0)),
            pl.BlockSpec((RB, KVW), lambda i: (i, 0)),
        ],
        out_shape=[
            jax.ShapeDtypeStruct((N, QW), F32),
            jax.ShapeDtypeStruct((N, KVW), F32),
        ],
    )(h, wqt, wkvt)


_SQ2I = np.float32(1.0 / np.sqrt(2.0))


def _post_body(p_ref, h_ref, expand_ref, wo_ref, bo_ref, w1_ref, b1_ref,
               w2_ref, b2_ref, l1g_ref, l1b_ref, l2g_ref, l2b_ref, o_ref):
    num = p_ref[0, :, :D] + p_ref[1, :, :D]
    den = p_ref[0, :, D:D + H] + p_ref[1, :, D:D + H]
    rec = 1.0 / (den + 1e-16)
    attn = num * jnp.dot(rec, expand_ref[...], preferred_element_type=F32)
    y = jnp.dot(attn, wo_ref[...], preferred_element_type=F32) + bo_ref[...] + h_ref[...]
    y = _ln(y, l1g_ref[...], l1b_ref[...])
    t = jnp.dot(y, w1_ref[...], preferred_element_type=F32) + b1_ref[...]
    t = 0.5 * t * (1.0 + lax.erf(t * _SQ2I))
    f = jnp.dot(t, w2_ref[...], preferred_element_type=F32) + b2_ref[...]
    o_ref[...] = _ln(y + f, l2g_ref[...], l2b_ref[...])


def _post(partials, h, expand, wot, bo, w1t, b1, w2t, b2, l1g, l1b, l2g, l2b):
    full = lambda a, b_: pl.BlockSpec((a, b_), lambda i: (0, 0))
    return pl.pallas_call(
        _post_body,
        grid=(GRID,),
        in_specs=[
            pl.BlockSpec((2, RB, D + 2 * H), lambda i: (0, i, 0)),
            pl.BlockSpec((RB, D), lambda i: (i, 0)),
            full(H, D), full(D, D), full(1, D),
            full(D, FF), full(1, FF), full(FF, D), full(1, D),
            full(1, D), full(1, D), full(1, D), full(1, D),
        ],
        out_specs=pl.BlockSpec((RB, D), lambda i: (i, 0)),
        out_shape=jax.ShapeDtypeStruct((N, D), F32),
    )(partials, h, expand, wot, bo, w1t, b1, w2t, b2, l1g, l1b, l2g, l2b)


def _fusion_body(h0_ref, h1_ref, h2_ref, h3_ref, fw_ref, fb_ref, g_ref, b_ref,
                 o_ref):
    s = jnp.dot(h0_ref[...], fw_ref[0:D, :], preferred_element_type=F32)
    s += jnp.dot(h1_ref[...], fw_ref[D:2 * D, :], preferred_element_type=F32)
    s += jnp.dot(h2_ref[...], fw_ref[2 * D:3 * D, :], preferred_element_type=F32)
    s += jnp.dot(h3_ref[...], fw_ref[3 * D:4 * D, :], preferred_element_type=F32)
    s += fb_ref[...]
    o_ref[...] = _ln(s, g_ref[...], b_ref[...])


def _fusion(hs, fwt, fb, g, b):
    full = lambda a, b_: pl.BlockSpec((a, b_), lambda i: (0, 0))
    return pl.pallas_call(
        _fusion_body,
        grid=(GRID,),
        in_specs=[pl.BlockSpec((RB, D), lambda i: (i, 0))] * 4 + [
            full(S * D, D), full(1, D), full(1, D), full(1, D)],
        out_specs=pl.BlockSpec((RB, D), lambda i: (i, 0)),
        out_shape=jax.ShapeDtypeStruct((N, D), F32),
    )(*hs, fwt, fb, g, b)


NC = 2
NS = 16
NW = NC * NS
C = 32
PW = D + 2 * H
SP = 512
CPS = SP // C
NSP = E // SP
SP_BASE = NSP // NW
SP_REM = NSP % NW
NP_ = 10240
RPT = NP_ // NS
_SCALE_ATTN = np.float32(1.0 / np.sqrt(DK))


def _edge_partials(q, kv, src, dst, scale, fcl, zblk):
    mesh = plsc.VectorSubcoreMesh(core_axis_name="c", subcore_axis_name="s")

    @functools.partial(
        pl.kernel,
        out_type=jax.ShapeDtypeStruct((NC, NP_, PW), F32),
        mesh=mesh,
        compiler_params=pltpu.CompilerParams(needs_layout_passes=False,
                                             use_tc_tiling_on_sc=False),
        scratch_types=[
            pltpu.VMEM((SP,), jnp.int32),
            pltpu.VMEM((CPS, C), jnp.int32),
            pltpu.VMEM((SP,), F32),
            pltpu.VMEM((SP,), F32),
            pltpu.VMEM((C, QW), F32),
            pltpu.VMEM((C, QW), F32),
            pltpu.VMEM((C, KVW), F32),
            pltpu.VMEM((C, KVW), F32),
            pltpu.VMEM((C, PW), F32),
            pltpu.VMEM_SHARED((NP_, PW), F32),
            pltpu.SemaphoreType.DMA,
            pltpu.SemaphoreType.DMA,
            pltpu.SemaphoreType.DMA,
            pltpu.SemaphoreType.DMA,
        ],
    )
    def edge_kernel(q_hbm, kv_hbm, src_hbm, dst2_hbm, scale_hbm, fcl_hbm,
                    z_hbm, out_hbm, src_v, dst3, scale_v, fcl_v, qrows0,
                    qrows1, kvrows0, kvrows1, wvex, acc, semq0, semq1,
                    semk0, semk1):
        cid = lax.axis_index("c")
        sid = lax.axis_index("s")
        wid = sid * NC + cid
        pltpu.sync_copy(z_hbm, wvex)
        row0 = sid * RPT
        for jz in range(RPT // C):
            pltpu.sync_copy(z_hbm, acc.at[pl.ds(row0 + jz * C, C)])
        plsc.subcore_barrier()

        nsp = jnp.where(wid < SP_REM, SP_BASE + 1, SP_BASE)

        qbufs = (qrows0, qrows1)
        kbufs = (kvrows0, kvrows1)
        qsems = (semq0, semq1)
        ksems = (semk0, semk1)

        def issue(cc, b):
            pltpu.async_copy(q_hbm.at[dst3.at[cc]], qbufs[b], qsems[b])
            pltpu.async_copy(kv_hbm.at[src_v.at[pl.ds(cc * C, C)]],
                             kbufs[b], ksems[b])

        def wait(b):
            pltpu.make_async_copy(q_hbm.at[pl.ds(0, C)], qbufs[b],
                                  qsems[b]).wait()
            pltpu.make_async_copy(kv_hbm.at[pl.ds(0, C)], kbufs[b],
                                  ksems[b]).wait()

        def compute(cc, b):
            qrows = qbufs[b]
            kvrows = kbufs[b]
            cb = cc * C

            @plsc.parallel_loop(0, C // 16, 1, unroll=1)
            def group(g):
                eidx = g * 16 + lax.iota(jnp.int32, 16)
                scl = scale_v[pl.ds(cb + g * 16, 16)]
                fcv = fcl_v[pl.ds(cb + g * 16, 16)]
                for h in range(H):
                    cbase = h * DK
                    part = [jnp.zeros((16,), F32) for _ in range(4)]
                    for j in range(DK):
                        colv = jnp.full((16,), cbase + j, jnp.int32)
                        qv = plsc.load_gather(qrows, [eidx, colv])
                        kj = plsc.load_gather(kvrows, [eidx, colv])
                        part[j % 4] = part[j % 4] + qv * kj
                    logit = (part[0] + part[1]) + (part[2] + part[3])
                    ex = jnp.exp(logit * scl + fcv)
                    plsc.store_scatter(
                        wvex, [eidx, jnp.full((16,), D + h, jnp.int32)], ex)

            @plsc.parallel_loop(0, C, 1, unroll=1)
            def vpass(e):
                exrow = wvex[e, pl.ds(D, 16)]
                for h in range(H):
                    exb = jnp.broadcast_to(exrow[h], (DK,))
                    vchunk = kvrows[e, pl.ds(D + h * DK, DK)]
                    wvex[e, pl.ds(h * DK, DK)] = exb * vchunk

            pltpu.sync_copy(wvex, acc.at[dst3.at[cc]], add=True)

        def sp_body(i, carry):
            t = wid + i * NW
            base = t * SP
            pltpu.sync_copy(src_hbm.at[pl.ds(base, SP)], src_v)
            pltpu.sync_copy(dst2_hbm.at[pl.ds(t * CPS, CPS)], dst3)
            pltpu.sync_copy(scale_hbm.at[pl.ds(base, SP)], scale_v)
            pltpu.sync_copy(fcl_hbm.at[pl.ds(base, SP)], fcl_v)
            issue(0, 0)

            def pair_body(c2, pcarry):
                for b in range(2):
                    cc = c2 * 2 + b

                    @pl.when(cc + 1 < CPS)
                    def _():
                        issue(cc + 1, 1 - b)

                    wait(b)
                    compute(cc, b)
                return pcarry

            lax.fori_loop(0, CPS // 2, pair_body, 0)
            return carry

        lax.fori_loop(0, nsp, sp_body, 0)
        plsc.subcore_barrier()
        for jz in range(RPT // C):
            pltpu.sync_copy(acc.at[pl.ds(row0 + jz * C, C)],
                            out_hbm.at[cid, pl.ds(row0 + jz * C, C)])

    return edge_kernel(q, kv, src, dst.reshape(E // C, C), scale, fcl, zblk)


def kernel(x, edge_index_list, sc_mask_list, fc_weights_list, input_proj_W,
           input_proj_b, stage_embed, WQ, WK, WV, WO, bO, ln1_g, ln1_b, ln2_g,
           ln2_b, W1, b1, W2, b2, fc_lambda, fusion_W, fusion_b, norm_g,
           norm_b):
    expand = jnp.kron(jnp.eye(H, dtype=F32), jnp.ones((1, DK), F32))
    ipwt = input_proj_W.T
    zblk = jnp.zeros((C, PW), F32)
    outs = []
    for k in range(S):
        bse = (input_proj_b + stage_embed[k]).reshape(1, D)
        h = _pre(x, ipwt, bse)
        src = edge_index_list[k, 0]
        dst = edge_index_list[k, 1]
        scale = sc_mask_list[k].astype(F32) * _SCALE_ATTN
        for l in range(L):
            q, kv = _qkv(h, WQ[l].T, jnp.concatenate([WK[l].T, WV[l].T], axis=1))
            partials = _edge_partials(q, kv, src, dst, scale,
                                      fc_lambda[l] * fc_weights_list[k], zblk)
            h = _post(partials, h, expand, WO[l].T, bO[l].reshape(1, D),
                      W1[l].T, b1[l].reshape(1, FF), W2[l].T,
                      b2[l].reshape(1, D), ln1_g[l].reshape(1, D),
                      ln1_b[l].reshape(1, D), ln2_g[l].reshape(1, D),
                      ln2_b[l].reshape(1, D))
        outs.append(h)
    return _fusion(outs, fusion_W.T, fusion_b.reshape(1, D),
                   norm_g.reshape(1, D), norm_b.reshape(1, D))


# speedup vs baseline: 3.5407x; 1.1587x over previous
import functools

import jax
import jax.numpy as jnp
import numpy as np
from jax import lax
from jax.experimental import pallas as pl
from jax.experimental.pallas import tpu as pltpu
from jax.experimental.pallas import tpu_sc as plsc

N = 10000
E = 320000
D = 128
L = 2
S = 4
FF = 512
H = 8
DK = 16
RB = 1000
GRID = N // RB
F32 = jnp.float32


def _ln(y, g, b):
    m = jnp.mean(y, -1, keepdims=True)
    v = jnp.mean((y - m) ** 2, -1, keepdims=True)
    return (y - m) * lax.rsqrt(v + 1e-5) * g + b


def _pre_body(x_ref, w_ref, bse_ref, o_ref):
    o_ref[...] = (jnp.dot(x_ref[...], w_ref[...], preferred_element_type=F32)
                  + bse_ref[...])


def _pre(x, wt, bse):
    return pl.pallas_call(
        _pre_body,
        grid=(GRID,),
        in_specs=[
            pl.BlockSpec((RB, D), lambda i: (i, 0)),
            pl.BlockSpec((D, D), lambda i: (0, 0)),
            pl.BlockSpec((1, D), lambda i: (0, 0)),
        ],
        out_specs=pl.BlockSpec((RB, D), lambda i: (i, 0)),
        out_shape=jax.ShapeDtypeStruct((N, D), F32),
    )(x, wt, bse)


QW = 136
KVW = 264


def _qkv_body(h_ref, wq_ref, wkv_ref, q_ref, kv_ref):
    h = h_ref[...]
    q_ref[:, :D] = jnp.dot(h, wq_ref[...], preferred_element_type=F32)
    kv_ref[:, :2 * D] = jnp.dot(h, wkv_ref[...], preferred_element_type=F32)


def _qkv(h, wqt, wkvt):
    return pl.pallas_call(
        _qkv_body,
        grid=(GRID,),
        in_specs=[
            pl.BlockSpec((RB, D), lambda i: (i, 0)),
            pl.BlockSpec((D, D), lambda i: (0, 0)),
            pl.BlockSpec((D, 2 * D), lambda i: (0, 0)),
        ],
        out_specs=[
            pl.BlockSpec((RB, QW), lambda i: (i, 0)),
            pl.BlockSpec((RB, KVW), lambda i: (i, 0)),
        ],
        out_shape=[
            jax.ShapeDtypeStruct((N, QW), F32),
            jax.ShapeDtypeStruct((N, KVW), F32),
        ],
    )(h, wqt, wkvt)


_SQ2I = np.float32(1.0 / np.sqrt(2.0))


def _post_body(p_ref, h_ref, expand_ref, wo_ref, bo_ref, w1_ref, b1_ref,
               w2_ref, b2_ref, l1g_ref, l1b_ref, l2g_ref, l2b_ref, o_ref):
    num = p_ref[0, :, :D] + p_ref[1, :, :D]
    den = p_ref[0, :, D:D + H] + p_ref[1, :, D:D + H]
    rec = 1.0 / (den + 1e-16)
    attn = num * jnp.dot(rec, expand_ref[...], preferred_element_type=F32)
    y = jnp.dot(attn, wo_ref[...], preferred_element_type=F32) + bo_ref[...] + h_ref[...]
    y = _ln(y, l1g_ref[...], l1b_ref[...])
    t = jnp.dot(y, w1_ref[...], preferred_element_type=F32) + b1_ref[...]
    t = 0.5 * t * (1.0 + lax.erf(t * _SQ2I))
    f = jnp.dot(t, w2_ref[...], preferred_element_type=F32) + b2_ref[...]
    o_ref[...] = _ln(y + f, l2g_ref[...], l2b_ref[...])


def _post(partials, h, expand, wot, bo, w1t, b1, w2t, b2, l1g, l1b, l2g, l2b):
    full = lambda a, b_: pl.BlockSpec((a, b_), lambda i: (0, 0))
    return pl.pallas_call(
        _post_body,
        grid=(GRID,),
        in_specs=[
            pl.BlockSpec((2, RB, D + 2 * H), lambda i: (0, i, 0)),
            pl.BlockSpec((RB, D), lambda i: (i, 0)),
            full(H, D), full(D, D), full(1, D),
            full(D, FF), full(1, FF), full(FF, D), full(1, D),
            full(1, D), full(1, D), full(1, D), full(1, D),
        ],
        out_specs=pl.BlockSpec((RB, D), lambda i: (i, 0)),
        out_shape=jax.ShapeDtypeStruct((N, D), F32),
    )(partials, h, expand, wot, bo, w1t, b1, w2t, b2, l1g, l1b, l2g, l2b)


def _fusion_body(h0_ref, h1_ref, h2_ref, h3_ref, fw_ref, fb_ref, g_ref, b_ref,
                 o_ref):
    s = jnp.dot(h0_ref[...], fw_ref[0:D, :], preferred_element_type=F32)
    s += jnp.dot(h1_ref[...], fw_ref[D:2 * D, :], preferred_element_type=F32)
    s += jnp.dot(h2_ref[...], fw_ref[2 * D:3 * D, :], preferred_element_type=F32)
    s += jnp.dot(h3_ref[...], fw_ref[3 * D:4 * D, :], preferred_element_type=F32)
    s += fb_ref[...]
    o_ref[...] = _ln(s, g_ref[...], b_ref[...])


def _fusion(hs, fwt, fb, g, b):
    full = lambda a, b_: pl.BlockSpec((a, b_), lambda i: (0, 0))
    return pl.pallas_call(
        _fusion_body,
        grid=(GRID,),
        in_specs=[pl.BlockSpec((RB, D), lambda i: (i, 0))] * 4 + [
            full(S * D, D), full(1, D), full(1, D), full(1, D)],
        out_specs=pl.BlockSpec((RB, D), lambda i: (i, 0)),
        out_shape=jax.ShapeDtypeStruct((N, D), F32),
    )(*hs, fwt, fb, g, b)


NC = 2
NS = 16
NW = NC * NS
C = 32
PW = D + 2 * H
SP = 512
CPS = SP // C
NSP = E // SP
SP_BASE = NSP // NW
SP_REM = NSP % NW
NP_ = 10240
RPT = NP_ // NS
_SCALE_ATTN = np.float32(1.0 / np.sqrt(DK))
_IOTA16 = np.arange(16)
_BITREV = np.array([0, 8, 4, 12, 2, 10, 6, 14, 1, 9, 5, 13, 3, 11, 7, 15])
_PIB = jax.lax.GatherScatterMode.PROMISE_IN_BOUNDS


def _edge_partials(q, kv, src, dst, scale, fcl, zblk):
    mesh = plsc.VectorSubcoreMesh(core_axis_name="c", subcore_axis_name="s")

    @functools.partial(
        pl.kernel,
        out_type=jax.ShapeDtypeStruct((NC, NP_, PW), F32),
        mesh=mesh,
        compiler_params=pltpu.CompilerParams(needs_layout_passes=False,
                                             use_tc_tiling_on_sc=False),
        scratch_types=[
            pltpu.VMEM((SP,), jnp.int32),
            pltpu.VMEM((CPS, C), jnp.int32),
            pltpu.VMEM((SP + 16,), F32),
            pltpu.VMEM((SP + 16,), F32),
            pltpu.VMEM((C, QW), F32),
            pltpu.VMEM((C, QW), F32),
            pltpu.VMEM((C, KVW), F32),
            pltpu.VMEM((C, KVW), F32),
            pltpu.VMEM((C, PW), F32),
            pltpu.VMEM_SHARED((NP_, PW), F32),
            pltpu.SemaphoreType.DMA,
            pltpu.SemaphoreType.DMA,
            pltpu.SemaphoreType.DMA,
            pltpu.SemaphoreType.DMA,
        ],
    )
    def edge_kernel(q_hbm, kv_hbm, src_hbm, dst2_hbm, scale_hbm, fcl_hbm,
                    z_hbm, out_hbm, src_v, dst3, scale_v, fcl_v, qrows0,
                    qrows1, kvrows0, kvrows1, wvex, acc, semq0, semq1,
                    semk0, semk1):
        cid = lax.axis_index("c")
        sid = lax.axis_index("s")
        wid = sid * NC + cid
        iota = lax.iota(jnp.int32, 16)
        bitrev = ((iota & 1) << 3) | ((iota & 2) << 1) | ((iota & 4) >> 1) | ((iota & 8) >> 3)
        m8 = iota < 8
        lo8 = iota & 7
        pltpu.sync_copy(z_hbm, wvex)
        row0 = sid * RPT
        for jz in range(RPT // C):
            pltpu.sync_copy(z_hbm, acc.at[pl.ds(row0 + jz * C, C)])
        plsc.subcore_barrier()

        nsp = jnp.where(wid < SP_REM, SP_BASE + 1, SP_BASE)

        qbufs = (qrows0, qrows1)
        kbufs = (kvrows0, kvrows1)
        qsems = (semq0, semq1)
        ksems = (semk0, semk1)

        def issue(cc, b):
            pltpu.async_copy(q_hbm.at[dst3.at[cc]], qbufs[b], qsems[b])
            pltpu.async_copy(kv_hbm.at[src_v.at[pl.ds(cc * C, C)]],
                             kbufs[b], ksems[b])

        def wait(b):
            pltpu.make_async_copy(q_hbm.at[pl.ds(0, C)], qbufs[b],
                                  qsems[b]).wait()
            pltpu.make_async_copy(kv_hbm.at[pl.ds(0, C)], kbufs[b],
                                  ksems[b]).wait()

        def compute(cc, b):
            qrows = qbufs[b]
            kvrows = kbufs[b]
            cb = cc * C

            @plsc.parallel_loop(0, C // 2, 1, unroll=1)
            def qkpair(p):
                e0 = p * 2
                e1 = e0 + 1
                vecs = []
                for e in (e0, e1):
                    for h in range(H):
                        qh = qrows[e, pl.ds(h * DK, DK)]
                        kh = kvrows[e, pl.ds(h * DK, DK)]
                        vecs.append(qh * kh)
                width = 16
                while len(vecs) > 1:
                    width //= 2
                    ixc = iota ^ width
                    vecs = [v + v.at[ixc].get(mode=_PIB) for v in vecs]
                    mk = (iota & width) == 0
                    vecs = [jnp.where(mk, vecs[2 * i], vecs[2 * i + 1])
                            for i in range(len(vecs) // 2)]
                logit = vecs[0].at[bitrev].get(mode=_PIB)
                sc2 = scale_v[pl.ds(cb + e0, 16)]
                fc2 = fcl_v[pl.ds(cb + e0, 16)]
                sclv = jnp.where(m8, jnp.broadcast_to(sc2[0], (16,)),
                                 jnp.broadcast_to(sc2[1], (16,)))
                fcv = jnp.where(m8, jnp.broadcast_to(fc2[0], (16,)),
                                jnp.broadcast_to(fc2[1], (16,)))
                ex = jnp.exp(logit * sclv + fcv)
                wvex[e0, pl.ds(D, 16)] = ex.at[lo8].get(mode=_PIB)
                wvex[e1, pl.ds(D, 16)] = ex.at[lo8 + 8].get(mode=_PIB)

            @plsc.parallel_loop(0, C, 1, unroll=1)
            def vpass(e):
                exrow = wvex[e, pl.ds(D, 16)]
                for h in range(H):
                    exb = jnp.broadcast_to(exrow[h], (DK,))
                    vchunk = kvrows[e, pl.ds(D + h * DK, DK)]
                    wvex[e, pl.ds(h * DK, DK)] = exb * vchunk

            pltpu.sync_copy(wvex, acc.at[dst3.at[cc]], add=True)

        def sp_body(i, carry):
            t = wid + i * NW
            base = t * SP
            pltpu.sync_copy(src_hbm.at[pl.ds(base, SP)], src_v)
            pltpu.sync_copy(dst2_hbm.at[pl.ds(t * CPS, CPS)], dst3)
            pltpu.sync_copy(scale_hbm.at[pl.ds(base, SP)],
                            scale_v.at[pl.ds(0, SP)])
            pltpu.sync_copy(fcl_hbm.at[pl.ds(base, SP)],
                            fcl_v.at[pl.ds(0, SP)])
            issue(0, 0)

            def pair_body(c2, pcarry):
                for b in range(2):
                    cc = c2 * 2 + b

                    @pl.when(cc + 1 < CPS)
                    def _():
                        issue(cc + 1, 1 - b)

                    wait(b)
                    compute(cc, b)
                return pcarry

            lax.fori_loop(0, CPS // 2, pair_body, 0)
            return carry

        lax.fori_loop(0, nsp, sp_body, 0)
        plsc.subcore_barrier()
        for jz in range(RPT // C):
            pltpu.sync_copy(acc.at[pl.ds(row0 + jz * C, C)],
                            out_hbm.at[cid, pl.ds(row0 + jz * C, C)])

    return edge_kernel(q, kv, src, dst.reshape(E // C, C), scale, fcl, zblk)


def kernel(x, edge_index_list, sc_mask_list, fc_weights_list, input_proj_W,
           input_proj_b, stage_embed, WQ, WK, WV, WO, bO, ln1_g, ln1_b, ln2_g,
           ln2_b, W1, b1, W2, b2, fc_lambda, fusion_W, fusion_b, norm_g,
           norm_b):
    expand = jnp.kron(jnp.eye(H, dtype=F32), jnp.ones((1, DK), F32))
    ipwt = input_proj_W.T
    zblk = jnp.zeros((C, PW), F32)
    outs = []
    for k in range(S):
        bse = (input_proj_b + stage_embed[k]).reshape(1, D)
        h = _pre(x, ipwt, bse)
        src = edge_index_list[k, 0]
        dst = edge_index_list[k, 1]
        scale = sc_mask_list[k].astype(F32) * _SCALE_ATTN
        for l in range(L):
            q, kv = _qkv(h, WQ[l].T, jnp.concatenate([WK[l].T, WV[l].T], axis=1))
            partials = _edge_partials(q, kv, src, dst, scale,
                                      fc_lambda[l] * fc_weights_list[k], zblk)
            h = _post(partials, h, expand, WO[l].T, bO[l].reshape(1, D),
                      W1[l].T, b1[l].reshape(1, FF), W2[l].T,
                      b2[l].reshape(1, D), ln1_g[l].reshape(1, D),
                      ln1_b[l].reshape(1, D), ln2_g[l].reshape(1, D),
                      ln2_b[l].reshape(1, D))
        outs.append(h)
    return _fusion(outs, fusion_W.T, fusion_b.reshape(1, D),
                   norm_g.reshape(1, D), norm_b.reshape(1, D))
